# Initial kernel scaffold; baseline (speedup 1.0000x reference)
import jax
import jax.numpy as jnp
from jax.experimental import pallas as pl


def kernel(inputs, pos_table):
    raise NotImplementedError("write your pallas kernel here")


# speedup vs baseline: 1.7198x; 1.7198x over previous
import jax
import jax.numpy as jnp
from jax.experimental import pallas as pl
from jax.experimental.pallas import tpu as pltpu


def _body(in_ref, tab_ref, out_ref):
    out_ref[...] = in_ref[...] + tab_ref[...][None, :, :]


def kernel(inputs, pos_table):
    B, S, D = inputs.shape
    SBLK = 256
    return pl.pallas_call(
        _body,
        grid=(S // SBLK,),
        in_specs=[
            pl.BlockSpec((B, SBLK, D), lambda i: (0, i, 0)),
            pl.BlockSpec((SBLK, D), lambda i: (i, 0)),
        ],
        out_specs=pl.BlockSpec((B, SBLK, D), lambda i: (0, i, 0)),
        out_shape=jax.ShapeDtypeStruct((B, S, D), jnp.float32),
        compiler_params=pltpu.CompilerParams(
            dimension_semantics=("arbitrary",),
        ),
    )(inputs, pos_table)
